# baseline (speedup 1.0000x reference)
import functools

import jax
import jax.numpy as jnp
from jax import lax
from jax.experimental import pallas as pl
from jax.experimental.pallas import tpu as pltpu
from jax.experimental.pallas import tpu_sc as plsc

BATCH = 16384
HIST = 200
D_FIELD = 16

BPW = 512
KT = 2
BT = 4
HC = 4
NCHUNK = HIST // HC
CROWS = HC * BPW
HPLANE = KT * 128 * 8 * 128
WBLK = 8 * 128 * BT


@functools.cache
def _build(n_batch, n_vocab):
    info = plsc.get_sparse_core_info()
    nc = info.num_cores

    mesh = plsc.VectorSubcoreMesh(core_axis_name="c", subcore_axis_name="s")

    @functools.partial(
        pl.kernel,
        mesh=mesh,
        out_type=jax.ShapeDtypeStruct((BATCH * HIST * D_FIELD,), jnp.float32),
        scratch_types=[
            pltpu.VMEM((2, CROWS), jnp.int32),
            pltpu.VMEM((2, CROWS, D_FIELD), jnp.float32),
            pltpu.VMEM((HC * KT * WBLK,), jnp.float32),
            pltpu.SemaphoreType.DMA((2,)),
            pltpu.SemaphoreType.DMA((2,)),
            pltpu.SemaphoreType.DMA((HC,)),
        ],
        compiler_params=pltpu.CompilerParams(
            use_tc_tiling_on_sc=False, needs_layout_passes=False
        ),
    )
    def gather_kernel(idx_hbm, table_hbm, out_hbm, idx_v, rows_v, stg_v, sem_i, sem_g, sem_o):
        wid = lax.axis_index("s") * nc + lax.axis_index("c")
        lane = lax.iota(jnp.int32, 16)

        def fetch_idx(tr, r0, slot):
            for tcl in range(BT):
                off = tr * 131072 + (BT * wid + tcl) * 1024 + r0 * 128
                pltpu.async_copy(
                    idx_hbm.at[pl.ds(off, HC * 128)],
                    idx_v.at[slot, pl.ds(tcl * HC * 128, HC * 128)],
                    sem_i.at[slot],
                )

        def wait_idx(slot):
            pltpu.make_async_copy(
                idx_hbm.at[pl.ds(0, CROWS)], idx_v.at[slot], sem_i.at[slot]
            ).wait()

        fetch_idx(0, 0, 0)
        fetch_idx(0, 4, 1)

        def transpose_h(j, hl, sr):
            rows2d = rows_v.at[sr]

            @pl.when(j >= 1)
            def _reclaim():
                pltpu.make_async_copy(
                    stg_v.at[pl.ds(0, KT * WBLK)],
                    out_hbm.at[pl.ds(0, KT * WBLK)],
                    sem_o.at[hl],
                ).wait()
            pat = (lane // 8) * WBLK + lax.rem(lane, 8) * 128
            for tcl in range(BT):
                for cb in range(8):
                    row0 = tcl * (HC * 128) + hl * 128 + cb * 16
                    vals = [rows2d[row0 + m] for m in range(16)]
                    base = hl * KT * WBLK + tcl * 1024 + cb * 16
                    for m in range(16):
                        plsc.store_scatter(
                            stg_v, [pat + (base + m)], vals[m]
                        )
            h = HC * j + hl
            for kt in range(KT):
                pltpu.async_copy(
                    stg_v.at[pl.ds(hl * KT * WBLK + kt * WBLK, WBLK)],
                    out_hbm.at[
                        pl.ds(h * HPLANE + kt * (HPLANE // KT) + wid * WBLK, WBLK)
                    ],
                    sem_o.at[hl],
                )

        def step(p, carry):
            for s in range(2):
                g = 2 * p + s

                @pl.when(g < NCHUNK)
                def _fire():
                    wait_idx(s)
                    pltpu.async_copy(
                        table_hbm.at[idx_v.at[s]], rows_v.at[s], sem_g.at[s]
                    )

                @pl.when((g >= 1) & (g <= NCHUNK))
                def _finish():
                    j = g - 1
                    sr = 1 - s
                    pltpu.make_async_copy(
                        table_hbm.at[idx_v.at[sr]], rows_v.at[sr], sem_g.at[sr]
                    ).wait()

                    @pl.when(j + 2 < NCHUNK)
                    def _prefetch_idx():
                        fetch_idx(p + s, 4 * sr, sr)

                    def hl_body(hl, c2):
                        transpose_h(j, hl, sr)
                        return c2

                    lax.fori_loop(0, HC, hl_body, 0)

            return carry

        lax.fori_loop(0, NCHUNK // 2 + 1, step, 0)

        for hl in range(HC):
            for kt in range(KT):
                pltpu.make_async_copy(
                    stg_v.at[pl.ds(0, WBLK)],
                    out_hbm.at[pl.ds(0, WBLK)],
                    sem_o.at[hl],
                ).wait()

    return gather_kernel


def kernel(lookup, table):
    idx_flat = (
        lookup.T.astype(jnp.int32)
        .reshape(25, 8, 128, 128)
        .transpose(0, 2, 1, 3)
        .reshape(BATCH * HIST)
    )
    flat = _build(lookup.shape[0], table.shape[0])(idx_flat, table)
    return (
        flat.reshape(HIST, KT, 128, 8, 128)
        .transpose(0, 1, 3, 2, 4)
        .reshape(HIST, D_FIELD, BATCH)
        .transpose(2, 0, 1)
    )


# speedup vs baseline: 1.8947x; 1.8947x over previous
import functools

import jax
import jax.numpy as jnp
from jax import lax
from jax.experimental import pallas as pl
from jax.experimental.pallas import tpu as pltpu
from jax.experimental.pallas import tpu_sc as plsc

BATCH = 16384
HIST = 200
D_FIELD = 16

BPW = 512
KT = 2
BT = 4
HC = 4
NCHUNK = HIST // HC
CROWS = HC * BPW
HPLANE = KT * 128 * 8 * 128
WBLK = 8 * 128 * BT


@functools.cache
def _build(n_batch, n_vocab):
    info = plsc.get_sparse_core_info()
    nc = info.num_cores

    mesh = plsc.VectorSubcoreMesh(core_axis_name="c", subcore_axis_name="s")

    @functools.partial(
        pl.kernel,
        mesh=mesh,
        out_type=jax.ShapeDtypeStruct((BATCH * HIST * D_FIELD,), jnp.float32),
        scratch_types=[
            pltpu.VMEM((2, CROWS), jnp.int32),
            pltpu.VMEM((2, CROWS, D_FIELD), jnp.float32),
            pltpu.VMEM((HC * KT * WBLK,), jnp.float32),
            pltpu.SemaphoreType.DMA((2,)),
            pltpu.SemaphoreType.DMA((2,)),
            pltpu.SemaphoreType.DMA((HC,)),
        ],
        compiler_params=pltpu.CompilerParams(
            use_tc_tiling_on_sc=False, needs_layout_passes=False
        ),
    )
    def gather_kernel(idx_hbm, table_hbm, out_hbm, idx_v, rows_v, stg_v, sem_i, sem_g, sem_o):
        wid = lax.axis_index("s") * nc + lax.axis_index("c")
        lane = lax.iota(jnp.int32, 16)
        cols = [lax.rem(lane + d, 16) for d in range(16)]
        diags = [
            (k // 8) * WBLK + lax.rem(k, 8) * 128 + lane
            for k in (lax.rem(lane + d, 16) for d in range(16))
        ]

        def fetch_idx(tr, r0, slot):
            for tcl in range(BT):
                off = tr * 131072 + (BT * wid + tcl) * 1024 + r0 * 128
                pltpu.async_copy(
                    idx_hbm.at[pl.ds(off, HC * 128)],
                    idx_v.at[slot, pl.ds(tcl * HC * 128, HC * 128)],
                    sem_i.at[slot],
                )

        def wait_idx(slot):
            pltpu.make_async_copy(
                idx_hbm.at[pl.ds(0, CROWS)], idx_v.at[slot], sem_i.at[slot]
            ).wait()

        fetch_idx(0, 0, 0)
        fetch_idx(0, 4, 1)

        def transpose_h(j, hl, sr):
            rows2d = rows_v.at[sr]

            @pl.when(j >= 1)
            def _reclaim():
                pltpu.make_async_copy(
                    stg_v.at[pl.ds(0, KT * WBLK)],
                    out_hbm.at[pl.ds(0, KT * WBLK)],
                    sem_o.at[hl],
                ).wait()
            for tcl in range(BT):
                for cb in range(8):
                    row0 = tcl * (HC * 128) + hl * 128 + cb * 16
                    base = hl * KT * WBLK + tcl * 1024 + cb * 16
                    for dh in range(2):
                        vs = [
                            plsc.load_gather(
                                rows2d, [lane + row0, cols[dh * 8 + d]]
                            )
                            for d in range(8)
                        ]
                        for d in range(8):
                            plsc.store_scatter(
                                stg_v, [diags[dh * 8 + d] + base], vs[d]
                            )
            h = HC * j + hl
            for kt in range(KT):
                pltpu.async_copy(
                    stg_v.at[pl.ds(hl * KT * WBLK + kt * WBLK, WBLK)],
                    out_hbm.at[
                        pl.ds(h * HPLANE + kt * (HPLANE // KT) + wid * WBLK, WBLK)
                    ],
                    sem_o.at[hl],
                )

        def step(p, carry):
            for s in range(2):
                g = 2 * p + s

                @pl.when(g < NCHUNK)
                def _fire():
                    wait_idx(s)
                    pltpu.async_copy(
                        table_hbm.at[idx_v.at[s]], rows_v.at[s], sem_g.at[s]
                    )

                @pl.when((g >= 1) & (g <= NCHUNK))
                def _finish():
                    j = g - 1
                    sr = 1 - s
                    pltpu.make_async_copy(
                        table_hbm.at[idx_v.at[sr]], rows_v.at[sr], sem_g.at[sr]
                    ).wait()

                    @pl.when(j + 2 < NCHUNK)
                    def _prefetch_idx():
                        fetch_idx(p + s, 4 * sr, sr)

                    def hl_body(hl, c2):
                        transpose_h(j, hl, sr)
                        return c2

                    lax.fori_loop(0, HC, hl_body, 0)

            return carry

        lax.fori_loop(0, NCHUNK // 2 + 1, step, 0)

        for hl in range(HC):
            for kt in range(KT):
                pltpu.make_async_copy(
                    stg_v.at[pl.ds(0, WBLK)],
                    out_hbm.at[pl.ds(0, WBLK)],
                    sem_o.at[hl],
                ).wait()

    return gather_kernel


def kernel(lookup, table):
    idx_flat = (
        lookup.T.astype(jnp.int32)
        .reshape(25, 8, 128, 128)
        .transpose(0, 2, 1, 3)
        .reshape(BATCH * HIST)
    )
    flat = _build(lookup.shape[0], table.shape[0])(idx_flat, table)
    return (
        flat.reshape(HIST, KT, 128, 8, 128)
        .transpose(0, 1, 3, 2, 4)
        .reshape(HIST, D_FIELD, BATCH)
        .transpose(2, 0, 1)
    )
